# baseline (speedup 1.0000x reference)
import functools

import jax
import jax.numpy as jnp
from jax import lax
from jax.experimental import pallas as pl
from jax.experimental.pallas import tpu as pltpu
from jax.experimental.pallas import tpu_sc as plsc

NUM_HEADS = 16
ROWS = 12
L = 16
NC, NS = 2, 16
NW = NC * NS
R = 8
RQ = 2
HH = 16


def _sc_gather_call(N):
    n_slabs = N // R
    per_w = n_slabs // NW
    n_grp = N // L
    tail_col = n_grp * L
    NQ = (R // RQ) * (NUM_HEADS // HH)

    mesh = plsc.VectorSubcoreMesh(
        core_axis_name="c", subcore_axis_name="s",
        num_cores=NC, num_subcores=NS)

    @functools.partial(
        pl.kernel,
        out_type=jax.ShapeDtypeStruct((N, NUM_HEADS, N), jnp.float32),
        mesh=mesh,
        compiler_params=pltpu.CompilerParams(needs_layout_passes=False),
        scratch_types=[
            pltpu.VMEM((2 * 128,), jnp.float32),
            pltpu.VMEM((2, R, N), jnp.int32),
            pltpu.VMEM((2, RQ, HH, N), jnp.float32),
            pltpu.SemaphoreType.DMA,
            pltpu.SemaphoreType.DMA,
            pltpu.SemaphoreType.DMA,
        ],
    )
    def body(w_hbm, idx_hbm, idxl_hbm, out_hbm, w_v, idx_v, out_v, sem_idx,
             s_out0, s_out1):
        wid = lax.axis_index("s") * NC + lax.axis_index("c")
        pltpu.sync_copy(w_hbm, w_v)

        lanes = lax.iota(jnp.int32, L)
        rows16 = lanes & (RQ - 1)
        col_t = jnp.full((L,), tail_col, jnp.int32)
        row_mask = lanes < RQ

        def drain(sem):
            pltpu.make_async_copy(
                out_v.at[0],
                out_hbm.at[pl.ds(0, RQ), pl.ds(0, HH), :], sem).wait()

        pltpu.async_copy(idx_hbm.at[pl.ds(wid * R, R), :], idx_v.at[0],
                         sem_idx)

        def do_slab(t, carry):
            ib = t & 1
            base = (wid + NW * t) * R
            pltpu.make_async_copy(idx_hbm.at[pl.ds(base, R), :],
                                  idx_v.at[ib], sem_idx).wait()

            @pl.when(t < per_w - 1)
            def _():
                nxt = (wid + NW * (t + 1)) * R
                pltpu.async_copy(idx_hbm.at[pl.ds(nxt, R), :],
                                 idx_v.at[(t + 1) & 1], sem_idx)

            def do_quarter(q, c1):
                n_half = NUM_HEADS // HH
                rsub = q // n_half
                half = q % n_half
                b = q & 1
                step = t * NQ + q

                @pl.when(step >= 2)
                def _():
                    @pl.when(b == 0)
                    def _():
                        drain(s_out0)

                    @pl.when(b == 1)
                    def _():
                        drain(s_out1)

                hbase = half * HH

                def row_body(r, c2):
                    for g in range(n_grp):
                        iv = idx_v[ib, rsub * RQ + r,
                                   pl.ds(g * L, L)] * NUM_HEADS
                        ivh = iv + hbase
                        for j in range(HH):
                            out_v[b, r, j, pl.ds(g * L, L)] = (
                                plsc.load_gather(w_v, [ivh + j]))
                    return c2
                lax.fori_loop(0, RQ, row_body, 0)

                tail_iv = plsc.load_gather(
                    idx_v.at[ib], [rsub * RQ + rows16, col_t]) * NUM_HEADS
                tail_ivh = tail_iv + hbase
                for j in range(HH):
                    tv = plsc.load_gather(w_v, [tail_ivh + j])
                    plsc.store_scatter(
                        out_v.at[b],
                        [rows16, jnp.full((L,), j, jnp.int32), col_t],
                        tv, mask=row_mask)

                dst = out_hbm.at[pl.ds(base + rsub * RQ, RQ),
                                 pl.ds(hbase, HH), :]

                @pl.when(b == 0)
                def _():
                    pltpu.async_copy(out_v.at[0], dst, s_out0)

                @pl.when(b == 1)
                def _():
                    pltpu.async_copy(out_v.at[1], dst, s_out1)
                return c1

            lax.fori_loop(0, NQ, do_quarter, 0)
            return carry

        lax.fori_loop(0, per_w, do_slab, 0)
        drain(s_out0)
        drain(s_out1)

        for half in range(NUM_HEADS // HH):
            @pl.when(wid == NW - 2 + half)
            def _():
                pltpu.sync_copy(idxl_hbm, idx_v.at[0])
                t16 = plsc.load_gather(
                    idx_v.at[0], [jnp.full((L,), R - 1, jnp.int32), col_t])
                t16 = t16 * NUM_HEADS

                def g_body(g, c2):
                    iv = idx_v[0, R - 1, pl.ds(g * L, L)] * NUM_HEADS
                    for j in range(HH):
                        out_v[0, 0, j, pl.ds(g * L, L)] = plsc.load_gather(
                            w_v, [iv + (HH * half + j)])
                    return c2
                lax.fori_loop(0, n_grp, g_body, 0)
                for j in range(HH):
                    tv = plsc.load_gather(w_v, [t16 + (HH * half + j)])
                    plsc.store_scatter(
                        out_v.at[0], [jnp.zeros((L,), jnp.int32),
                                      jnp.full((L,), j, jnp.int32), col_t],
                        tv, mask=lanes < 1)
                pltpu.sync_copy(
                    out_v.at[0, pl.ds(0, 1), :, :],
                    out_hbm.at[pl.ds(N - 1, 1), pl.ds(half * HH, HH), :])

    return body


def kernel(W, edge_bias):
    N = edge_bias.shape[0]
    call = _sc_gather_call(N)
    w_flat = jnp.pad(W.astype(jnp.float32).reshape(-1),
                     (0, 2 * 128 - ROWS * NUM_HEADS))
    idx_last = edge_bias[N - R:N]
    out = call(w_flat, edge_bias.astype(jnp.int32),
               idx_last.astype(jnp.int32))
    return jnp.transpose(out, (1, 0, 2))[None]


# speedup vs baseline: 1.0084x; 1.0084x over previous
import functools

import jax
import jax.numpy as jnp
from jax import lax
from jax.experimental import pallas as pl
from jax.experimental.pallas import tpu as pltpu
from jax.experimental.pallas import tpu_sc as plsc

NUM_HEADS = 16
ROWS = 12
L = 16
NC, NS = 2, 16
NW = NC * NS
R = 8
RQ = 4
HH = 8


def _sc_gather_call(N):
    n_slabs = N // R
    per_w = n_slabs // NW
    n_grp = N // L
    tail_col = n_grp * L
    NQ = (R // RQ) * (NUM_HEADS // HH)

    mesh = plsc.VectorSubcoreMesh(
        core_axis_name="c", subcore_axis_name="s",
        num_cores=NC, num_subcores=NS)

    @functools.partial(
        pl.kernel,
        out_type=jax.ShapeDtypeStruct((N, NUM_HEADS, N), jnp.float32),
        mesh=mesh,
        compiler_params=pltpu.CompilerParams(needs_layout_passes=False),
        scratch_types=[
            pltpu.VMEM((2 * 128,), jnp.float32),
            pltpu.VMEM((2, R, N), jnp.int32),
            pltpu.VMEM((2, RQ, HH, N), jnp.float32),
            pltpu.SemaphoreType.DMA,
            pltpu.SemaphoreType.DMA,
            pltpu.SemaphoreType.DMA,
        ],
    )
    def body(w_hbm, idx_hbm, idxl_hbm, out_hbm, w_v, idx_v, out_v, sem_idx,
             s_out0, s_out1):
        wid = lax.axis_index("s") * NC + lax.axis_index("c")
        pltpu.sync_copy(w_hbm, w_v)

        lanes = lax.iota(jnp.int32, L)
        rows16 = lanes & (RQ - 1)
        col_t = jnp.full((L,), tail_col, jnp.int32)
        row_mask = lanes < RQ

        def drain(sem):
            pltpu.make_async_copy(
                out_v.at[0],
                out_hbm.at[pl.ds(0, RQ), pl.ds(0, HH), :], sem).wait()

        pltpu.async_copy(idx_hbm.at[pl.ds(wid * R, R), :], idx_v.at[0],
                         sem_idx)

        def do_slab(t, carry):
            ib = t & 1
            base = (wid + NW * t) * R
            pltpu.make_async_copy(idx_hbm.at[pl.ds(base, R), :],
                                  idx_v.at[ib], sem_idx).wait()

            @pl.when(t < per_w - 1)
            def _():
                nxt = (wid + NW * (t + 1)) * R
                pltpu.async_copy(idx_hbm.at[pl.ds(nxt, R), :],
                                 idx_v.at[(t + 1) & 1], sem_idx)

            def do_quarter(q, c1):
                rsub = q >> 1
                half = q & 1
                b = q & 1
                step = t * NQ + q

                @pl.when(step >= 2)
                def _():
                    @pl.when(b == 0)
                    def _():
                        drain(s_out0)

                    @pl.when(b == 1)
                    def _():
                        drain(s_out1)

                hbase = half * HH

                def row_body(r, c2):
                    for g in range(n_grp):
                        iv = idx_v[ib, rsub * RQ + r,
                                   pl.ds(g * L, L)] * NUM_HEADS
                        ivh = iv + hbase
                        for j in range(HH):
                            out_v[b, r, j, pl.ds(g * L, L)] = (
                                plsc.load_gather(w_v, [ivh + j]))
                    return c2
                lax.fori_loop(0, RQ, row_body, 0)

                tail_iv = plsc.load_gather(
                    idx_v.at[ib], [rsub * RQ + rows16, col_t]) * NUM_HEADS
                tail_ivh = tail_iv + hbase
                for j in range(HH):
                    tv = plsc.load_gather(w_v, [tail_ivh + j])
                    plsc.store_scatter(
                        out_v.at[b],
                        [rows16, jnp.full((L,), j, jnp.int32), col_t],
                        tv, mask=row_mask)

                dst = out_hbm.at[pl.ds(base + rsub * RQ, RQ),
                                 pl.ds(hbase, HH), :]

                @pl.when(b == 0)
                def _():
                    pltpu.async_copy(out_v.at[0], dst, s_out0)

                @pl.when(b == 1)
                def _():
                    pltpu.async_copy(out_v.at[1], dst, s_out1)
                return c1

            lax.fori_loop(0, NQ, do_quarter, 0)
            return carry

        lax.fori_loop(0, per_w, do_slab, 0)
        drain(s_out0)
        drain(s_out1)

        for half in range(NUM_HEADS // HH):
            @pl.when(wid == NW - 2 + half)
            def _():
                pltpu.sync_copy(idxl_hbm, idx_v.at[0])
                t16 = plsc.load_gather(
                    idx_v.at[0], [jnp.full((L,), R - 1, jnp.int32), col_t])
                t16 = t16 * NUM_HEADS

                def g_body(g, c2):
                    iv = idx_v[0, R - 1, pl.ds(g * L, L)] * NUM_HEADS
                    for j in range(HH):
                        out_v[0, 0, j, pl.ds(g * L, L)] = plsc.load_gather(
                            w_v, [iv + (HH * half + j)])
                    return c2
                lax.fori_loop(0, n_grp, g_body, 0)
                for j in range(HH):
                    tv = plsc.load_gather(w_v, [t16 + (HH * half + j)])
                    plsc.store_scatter(
                        out_v.at[0], [jnp.zeros((L,), jnp.int32),
                                      jnp.full((L,), j, jnp.int32), col_t],
                        tv, mask=lanes < 1)
                pltpu.sync_copy(
                    out_v.at[0, pl.ds(0, 1), :, :],
                    out_hbm.at[pl.ds(N - 1, 1), pl.ds(half * HH, HH), :])

    return body


def kernel(W, edge_bias):
    N = edge_bias.shape[0]
    call = _sc_gather_call(N)
    w_flat = jnp.pad(W.astype(jnp.float32).reshape(-1),
                     (0, 2 * 128 - ROWS * NUM_HEADS))
    idx_last = edge_bias[N - R:N]
    out = call(w_flat, edge_bias.astype(jnp.int32),
               idx_last.astype(jnp.int32))
    return jnp.transpose(out, (1, 0, 2))[None]
